# baseline (speedup 1.0000x reference)
import functools

import jax
import jax.numpy as jnp
from jax import lax
from jax.experimental import pallas as pl
from jax.experimental.pallas import tpu as pltpu
from jax.experimental.pallas import tpu_sc as plsc

NUM_SEG = 64
D = 512
N = 50000
LANES = 16
NC = 2
NS = 16
NW = NC * NS
BLK_R = 80
NBLK = N // BLK_R
BPT = -(-NBLK // NW)
TAIL = NBLK - (NW - 1) * BPT
NVREG = D // LANES

_mesh = plsc.VectorSubcoreMesh(core_axis_name="c", subcore_axis_name="s")


@functools.partial(
    pl.kernel,
    mesh=_mesh,
    out_type=(
        jax.ShapeDtypeStruct((NW, NUM_SEG, D), jnp.float32),
        jax.ShapeDtypeStruct((NW, NUM_SEG, LANES), jnp.float32),
    ),
    scratch_types=[
        pltpu.VMEM((2, BLK_R, D), jnp.float32),
        pltpu.VMEM((2, BLK_R), jnp.int32),
        pltpu.VMEM((NUM_SEG, D), jnp.float32),
        pltpu.VMEM((NUM_SEG, LANES), jnp.float32),
        pltpu.SemaphoreType.DMA((2,)),
        pltpu.SemaphoreType.DMA((2,)),
        pltpu.SemaphoreType.DMA,
    ],
)
def _sc_seg_sum(x_hbm, b_hbm, psum_hbm, pcnt_hbm,
                xbuf, bbuf, acc, cnt, semx, semb, semo):
    wid = lax.axis_index("s") * NC + lax.axis_index("c")
    blk0 = wid * BPT

    zeros16 = jnp.zeros((LANES,), jnp.float32)
    ones16 = jnp.ones((LANES,), jnp.float32)

    @pl.loop(0, NUM_SEG)
    def _zero(r):
        for j in range(NVREG):
            acc[r, pl.ds(j * LANES, LANES)] = zeros16
        cnt[r, pl.ds(0, LANES)] = zeros16

    def x_copy(i, b):
        return pltpu.make_async_copy(
            x_hbm.at[pl.ds((blk0 + i) * BLK_R, BLK_R)], xbuf.at[b],
            semx.at[b])

    def b_copy(i, b):
        return pltpu.make_async_copy(
            b_hbm.at[pl.ds((blk0 + i) * BLK_R, BLK_R)], bbuf.at[b],
            semb.at[b])

    def process(b):
        @pl.loop(0, BLK_R, step=LANES)
        def _rows(r0):
            svec = bbuf[b, pl.ds(r0, LANES)]
            for k in range(LANES):
                s = svec[k]
                for j in range(NVREG):
                    sl = pl.ds(j * LANES, LANES)
                    plsc.addupdate(acc.at[s, sl], xbuf[b, r0 + k, sl])
                plsc.addupdate(cnt.at[s, pl.ds(0, LANES)], ones16)

    nblocks = jnp.where(wid == NW - 1, TAIL, BPT)

    x_copy(0, 0).start()
    b_copy(0, 0).start()
    x_copy(1, 1).start()
    b_copy(1, 1).start()

    @pl.loop(0, nblocks // 2)
    def _pairs(p):
        for b in (0, 1):
            i = 2 * p + b
            x_copy(i, b).wait()
            b_copy(i, b).wait()
            process(b)

            @pl.when(i + 2 < nblocks)
            def _pref():
                x_copy(i + 2, b).start()
                b_copy(i + 2, b).start()

    @pl.when(nblocks % 2 == 1)
    def _odd():
        i = nblocks - 1
        x_copy(i, 0).wait()
        b_copy(i, 0).wait()
        process(0)

    pltpu.make_async_copy(acc, psum_hbm.at[wid], semo).start()
    pltpu.make_async_copy(cnt, pcnt_hbm.at[wid], semo).start()
    pltpu.make_async_copy(acc, psum_hbm.at[wid], semo).wait()
    pltpu.make_async_copy(cnt, pcnt_hbm.at[wid], semo).wait()


def _combine_body(ps_ref, pc_ref, out_ref):
    sums = jnp.sum(ps_ref[...], axis=0)
    counts = jnp.sum(pc_ref[...], axis=0)
    out_ref[...] = sums / jnp.clip(counts[:, 0:1], 1.0, None)


def _tc_combine(psum, pcnt):
    return pl.pallas_call(
        _combine_body,
        out_shape=jax.ShapeDtypeStruct((NUM_SEG, D), jnp.float32),
    )(psum, pcnt)


@jax.jit
def kernel(x, batch):
    batch32 = batch.astype(jnp.int32)
    psum, pcnt = _sc_seg_sum(x, batch32)
    return _tc_combine(psum, pcnt)


# speedup vs baseline: 2.6792x; 2.6792x over previous
import dataclasses
import functools

import jax
import jax.numpy as jnp
from jax import lax
from jax.experimental import pallas as pl
from jax.experimental.pallas import tpu as pltpu
from jax.experimental.pallas import tpu_sc as plsc

NUM_SEG = 64
D = 512
N = 50000
LANES = 16
NC = 2
NS = 16
NW = NC * NS
SUP = 80
NSUP = N // SUP
BPT = -(-NSUP // NW)
TAIL = NSUP - (NW - 1) * BPT
NVREG = D // LANES

_mesh = plsc.VectorSubcoreMesh(core_axis_name="c", subcore_axis_name="s")

_sc_params = pltpu.CompilerParams()
if "needs_layout_passes" in pltpu.CompilerParams.__dataclass_fields__:
    _sc_params = dataclasses.replace(_sc_params, needs_layout_passes=False)


@functools.partial(
    pl.kernel,
    mesh=_mesh,
    compiler_params=_sc_params,
    out_type=(
        jax.ShapeDtypeStruct((NW, NUM_SEG, D), jnp.float32),
        jax.ShapeDtypeStruct((NW, NUM_SEG, LANES), jnp.float32),
    ),
    scratch_types=[
        pltpu.VMEM((2 * SUP, D), jnp.float32),
        pltpu.VMEM((2 * SUP,), jnp.int32),
        pltpu.VMEM((NUM_SEG, D), jnp.float32),
        pltpu.VMEM((NUM_SEG, LANES), jnp.float32),
        pltpu.SemaphoreType.DMA,
        pltpu.SemaphoreType.DMA,
        pltpu.SemaphoreType.DMA,
    ],
)
def _sc_seg_sum(x_hbm, b_hbm, psum_hbm, pcnt_hbm,
                xbuf, bbuf, acc, cnt, semx, semb, semo):
    wid = lax.axis_index("s") * NC + lax.axis_index("c")
    blk0 = wid * BPT
    nblocks = jnp.where(wid == NW - 1, TAIL, BPT)

    zeros16 = jnp.zeros((LANES,), jnp.float32)
    ones16 = jnp.ones((LANES,), jnp.float32)
    sixteen16 = jnp.full((LANES,), 16.0, jnp.float32)

    @pl.loop(0, NUM_SEG)
    def _zero(r):
        for j in range(NVREG):
            acc[r, pl.ds(j * LANES, LANES)] = zeros16
        cnt[r, pl.ds(0, LANES)] = zeros16

    def x_copy(i, base):
        return pltpu.make_async_copy(
            x_hbm.at[pl.ds((blk0 + i) * SUP, SUP)],
            xbuf.at[pl.ds(base, SUP)], semx)

    def b_copy(i, base):
        return pltpu.make_async_copy(
            b_hbm.at[pl.ds((blk0 + i) * SUP, SUP)],
            bbuf.at[pl.ds(base, SUP)], semb)

    x_copy(0, 0).start()
    b_copy(0, 0).start()
    x_copy(1, SUP).start()
    b_copy(1, SUP).start()

    @pl.loop(0, nblocks)
    def _blocks(p):
        base = (p % 2) * SUP
        x_copy(0, 0).wait()
        b_copy(0, 0).wait()

        @pl.loop(0, SUP, step=LANES)
        def _group(goff):
            r0 = base + goff
            svec = bbuf[pl.ds(r0, LANES)]
            smin = lax.reduce_min(svec, (0,))
            smax = lax.reduce_max(svec, (0,))

            @pl.when(smin == smax)
            def _uniform():
                for j in range(NVREG):
                    sl = pl.ds(j * LANES, LANES)
                    v = xbuf[r0, sl]
                    vals = [xbuf[r0 + k, sl] for k in range(1, LANES)]
                    vals.append(v)
                    while len(vals) > 1:
                        nxt = [vals[i] + vals[i + 1]
                               for i in range(0, len(vals) - 1, 2)]
                        if len(vals) % 2:
                            nxt.append(vals[-1])
                        vals = nxt
                    plsc.addupdate(acc.at[smin, sl], vals[0])
                plsc.addupdate(cnt.at[smin, pl.ds(0, LANES)], sixteen16)

            @pl.when(smin != smax)
            def _boundary():
                for k in range(LANES):
                    s = svec[k]
                    for j in range(NVREG):
                        sl = pl.ds(j * LANES, LANES)
                        plsc.addupdate(acc.at[s, sl], xbuf[r0 + k, sl])
                    plsc.addupdate(cnt.at[s, pl.ds(0, LANES)], ones16)

        @pl.when(p + 2 < nblocks)
        def _prefetch():
            x_copy(p + 2, base).start()
            b_copy(p + 2, base).start()

    pltpu.make_async_copy(acc, psum_hbm.at[wid], semo).start()
    pltpu.make_async_copy(cnt, pcnt_hbm.at[wid], semo).start()
    pltpu.make_async_copy(acc, psum_hbm.at[wid], semo).wait()
    pltpu.make_async_copy(cnt, pcnt_hbm.at[wid], semo).wait()


def _combine_body(ps_ref, pc_ref, out_ref):
    sums = jnp.sum(ps_ref[...], axis=0)
    counts = jnp.sum(pc_ref[...], axis=0)
    out_ref[...] = sums / jnp.clip(counts[:, 0:1], 1.0, None)


def _tc_combine(psum, pcnt):
    return pl.pallas_call(
        _combine_body,
        out_shape=jax.ShapeDtypeStruct((NUM_SEG, D), jnp.float32),
    )(psum, pcnt)


@jax.jit
def kernel(x, batch):
    batch32 = batch.astype(jnp.int32)
    psum, pcnt = _sc_seg_sum(x, batch32)
    return _tc_combine(psum, pcnt)


# speedup vs baseline: 3.9797x; 1.4854x over previous
import dataclasses
import functools

import jax
import jax.numpy as jnp
from jax import lax
from jax.experimental import pallas as pl
from jax.experimental.pallas import tpu as pltpu
from jax.experimental.pallas import tpu_sc as plsc

NUM_SEG = 64
D = 512
N = 50000
LANES = 16
NC = 2
NS = 16
NW = NC * NS
SUP = 80
NVREG = D // LANES

N_SC = 20000
N_TC = N - N_SC
NSUP = N_SC // SUP
BPT = -(-NSUP // NW)
TAIL = NSUP - (NW - 1) * BPT

R_TC = 1000
SKIP_TC = N_SC // R_TC
GRID_TC = N_TC // R_TC

_mesh = plsc.VectorSubcoreMesh(core_axis_name="c", subcore_axis_name="s")

_sc_params = pltpu.CompilerParams()
if "needs_layout_passes" in pltpu.CompilerParams.__dataclass_fields__:
    _sc_params = dataclasses.replace(_sc_params, needs_layout_passes=False)


@functools.partial(
    pl.kernel,
    mesh=_mesh,
    compiler_params=_sc_params,
    out_type=(
        jax.ShapeDtypeStruct((NW, NUM_SEG, D), jnp.float32),
        jax.ShapeDtypeStruct((NW, NUM_SEG, LANES), jnp.float32),
    ),
    scratch_types=[
        pltpu.VMEM((2 * SUP, D), jnp.float32),
        pltpu.VMEM((2 * SUP,), jnp.int32),
        pltpu.VMEM((NUM_SEG, D), jnp.float32),
        pltpu.VMEM((NUM_SEG, LANES), jnp.float32),
        pltpu.SemaphoreType.DMA,
        pltpu.SemaphoreType.DMA,
        pltpu.SemaphoreType.DMA,
    ],
)
def _sc_seg_sum(x_hbm, b_hbm, psum_hbm, pcnt_hbm,
                xbuf, bbuf, acc, cnt, semx, semb, semo):
    wid = lax.axis_index("s") * NC + lax.axis_index("c")
    blk0 = wid * BPT
    nblocks = jnp.where(wid == NW - 1, TAIL, BPT)

    zeros16 = jnp.zeros((LANES,), jnp.float32)
    ones16 = jnp.ones((LANES,), jnp.float32)
    sixteen16 = jnp.full((LANES,), 16.0, jnp.float32)

    @pl.loop(0, NUM_SEG)
    def _zero(r):
        for j in range(NVREG):
            acc[r, pl.ds(j * LANES, LANES)] = zeros16
        cnt[r, pl.ds(0, LANES)] = zeros16

    def x_copy(i, base):
        return pltpu.make_async_copy(
            x_hbm.at[pl.ds((blk0 + i) * SUP, SUP)],
            xbuf.at[pl.ds(base, SUP)], semx)

    def b_copy(i, base):
        return pltpu.make_async_copy(
            b_hbm.at[pl.ds((blk0 + i) * SUP, SUP)],
            bbuf.at[pl.ds(base, SUP)], semb)

    x_copy(0, 0).start()
    b_copy(0, 0).start()
    x_copy(1, SUP).start()
    b_copy(1, SUP).start()

    @pl.loop(0, nblocks)
    def _blocks(p):
        base = (p % 2) * SUP
        x_copy(0, 0).wait()
        b_copy(0, 0).wait()

        @pl.loop(0, SUP, step=LANES)
        def _group(goff):
            r0 = base + goff
            svec = bbuf[pl.ds(r0, LANES)]
            smin = lax.reduce_min(svec, (0,))
            smax = lax.reduce_max(svec, (0,))

            @pl.when(smin == smax)
            def _uniform():
                for j in range(NVREG):
                    sl = pl.ds(j * LANES, LANES)
                    vals = [xbuf[r0 + k, sl] for k in range(LANES)]
                    while len(vals) > 1:
                        nxt = [vals[i] + vals[i + 1]
                               for i in range(0, len(vals) - 1, 2)]
                        if len(vals) % 2:
                            nxt.append(vals[-1])
                        vals = nxt
                    plsc.addupdate(acc.at[smin, sl], vals[0])
                plsc.addupdate(cnt.at[smin, pl.ds(0, LANES)], sixteen16)

            @pl.when(smin != smax)
            def _boundary():
                for k in range(LANES):
                    s = svec[k]
                    for j in range(NVREG):
                        sl = pl.ds(j * LANES, LANES)
                        plsc.addupdate(acc.at[s, sl], xbuf[r0 + k, sl])
                    plsc.addupdate(cnt.at[s, pl.ds(0, LANES)], ones16)

        @pl.when(p + 2 < nblocks)
        def _prefetch():
            x_copy(p + 2, base).start()
            b_copy(p + 2, base).start()

    pltpu.make_async_copy(acc, psum_hbm.at[wid], semo).start()
    pltpu.make_async_copy(cnt, pcnt_hbm.at[wid], semo).start()
    pltpu.make_async_copy(acc, psum_hbm.at[wid], semo).wait()
    pltpu.make_async_copy(cnt, pcnt_hbm.at[wid], semo).wait()


def _tc_body(batch_ref, x_ref, sum_ref, cnt_ref):
    i = pl.program_id(0)

    @pl.when(i == 0)
    def _init():
        sum_ref[...] = jnp.zeros_like(sum_ref)
        cnt_ref[...] = jnp.zeros_like(cnt_ref)

    b = batch_ref[0, 0, :]
    onehot = (b[:, None] == jax.lax.broadcasted_iota(
        jnp.int32, (R_TC, NUM_SEG), 1)).astype(jnp.bfloat16)
    xb = x_ref[...].astype(jnp.bfloat16)
    sum_ref[...] += jax.lax.dot_general(
        onehot, xb, (((0,), (0,)), ((), ())),
        preferred_element_type=jnp.float32)
    cnt_ref[...] += jax.lax.dot_general(
        onehot, jnp.ones((R_TC, 128), jnp.bfloat16),
        (((0,), (0,)), ((), ())),
        preferred_element_type=jnp.float32)


def _tc_partial(x, batch3):
    return pl.pallas_call(
        _tc_body,
        grid=(GRID_TC,),
        in_specs=[
            pl.BlockSpec((1, 1, R_TC), lambda i: (i + SKIP_TC, 0, 0)),
            pl.BlockSpec((R_TC, D), lambda i: (i + SKIP_TC, 0)),
        ],
        out_specs=[
            pl.BlockSpec((NUM_SEG, D), lambda i: (0, 0)),
            pl.BlockSpec((NUM_SEG, 128), lambda i: (0, 0)),
        ],
        out_shape=[
            jax.ShapeDtypeStruct((NUM_SEG, D), jnp.float32),
            jax.ShapeDtypeStruct((NUM_SEG, 128), jnp.float32),
        ],
        compiler_params=pltpu.CompilerParams(
            dimension_semantics=("arbitrary",)),
    )(batch3, x)


def _combine_body(ps_ref, pc_ref, ts_ref, tcnt_ref, out_ref):
    sums = jnp.sum(ps_ref[...], axis=0) + ts_ref[...]
    counts = jnp.sum(pc_ref[...], axis=0)[:, 0:1] + tcnt_ref[:, 0:1]
    out_ref[...] = sums / jnp.clip(counts, 1.0, None)


def _tc_combine(psum, pcnt, tsum, tcnt):
    return pl.pallas_call(
        _combine_body,
        out_shape=jax.ShapeDtypeStruct((NUM_SEG, D), jnp.float32),
    )(psum, pcnt, tsum, tcnt)


@jax.jit
def kernel(x, batch):
    batch32 = batch.astype(jnp.int32)
    batch3 = batch32.reshape(N // R_TC, 1, R_TC)
    psum, pcnt = _sc_seg_sum(x, batch32)
    tsum, tcnt = _tc_partial(x, batch3)
    return _tc_combine(psum, pcnt, tsum, tcnt)


# speedup vs baseline: 3.9837x; 1.0010x over previous
import dataclasses
import functools

import jax
import jax.numpy as jnp
from jax import lax
from jax.experimental import pallas as pl
from jax.experimental.pallas import tpu as pltpu
from jax.experimental.pallas import tpu_sc as plsc

NUM_SEG = 64
D = 512
N = 50000
LANES = 16
NC = 2
NS = 16
NW = NC * NS
SUP = 80
NVREG = D // LANES

N_SC = 20000
N_TC = N - N_SC
NSUP = N_SC // SUP
BPT = -(-NSUP // NW)
TAIL = NSUP - (NW - 1) * BPT

R_TC = 1000
SKIP_TC = N_SC // R_TC
GRID_TC = N_TC // R_TC

_mesh = plsc.VectorSubcoreMesh(core_axis_name="c", subcore_axis_name="s")

_sc_params = pltpu.CompilerParams()
if "needs_layout_passes" in pltpu.CompilerParams.__dataclass_fields__:
    _sc_params = dataclasses.replace(_sc_params, needs_layout_passes=False)


@functools.partial(
    pl.kernel,
    mesh=_mesh,
    compiler_params=_sc_params,
    out_type=(
        jax.ShapeDtypeStruct((NW, NUM_SEG, D), jnp.float32),
        jax.ShapeDtypeStruct((NW, NUM_SEG, LANES), jnp.float32),
    ),
    scratch_types=[
        pltpu.VMEM((2 * SUP, D), jnp.float32),
        pltpu.VMEM((2 * SUP,), jnp.int32),
        pltpu.VMEM((NUM_SEG, D), jnp.float32),
        pltpu.VMEM((NUM_SEG, LANES), jnp.float32),
        pltpu.SemaphoreType.DMA,
        pltpu.SemaphoreType.DMA,
        pltpu.SemaphoreType.DMA,
    ],
)
def _sc_seg_sum(x_hbm, b_hbm, psum_hbm, pcnt_hbm,
                xbuf, bbuf, acc, cnt, semx, semb, semo):
    wid = lax.axis_index("s") * NC + lax.axis_index("c")
    blk0 = wid * BPT
    nblocks = jnp.where(wid == NW - 1, TAIL, BPT)

    zeros16 = jnp.zeros((LANES,), jnp.float32)
    ones16 = jnp.ones((LANES,), jnp.float32)
    sixteen16 = jnp.full((LANES,), 16.0, jnp.float32)

    @pl.loop(0, NUM_SEG)
    def _zero(r):
        for j in range(NVREG):
            acc[r, pl.ds(j * LANES, LANES)] = zeros16
        cnt[r, pl.ds(0, LANES)] = zeros16

    def x_copy(i, base):
        return pltpu.make_async_copy(
            x_hbm.at[pl.ds((blk0 + i) * SUP, SUP)],
            xbuf.at[pl.ds(base, SUP)], semx)

    def b_copy(i, base):
        return pltpu.make_async_copy(
            b_hbm.at[pl.ds((blk0 + i) * SUP, SUP)],
            bbuf.at[pl.ds(base, SUP)], semb)

    x_copy(0, 0).start()
    b_copy(0, 0).start()
    x_copy(1, SUP).start()
    b_copy(1, SUP).start()

    @pl.loop(0, nblocks)
    def _blocks(p):
        base = (p % 2) * SUP
        x_copy(0, 0).wait()
        b_copy(0, 0).wait()

        @pl.loop(0, SUP, step=LANES)
        def _group(goff):
            r0 = base + goff
            svec = bbuf[pl.ds(r0, LANES)]
            smin = lax.reduce_min(svec, (0,))
            smax = lax.reduce_max(svec, (0,))

            @pl.when(smin == smax)
            def _uniform():
                for j in range(NVREG):
                    sl = pl.ds(j * LANES, LANES)
                    vals = [xbuf[r0 + k, sl] for k in range(LANES)]
                    while len(vals) > 1:
                        nxt = [vals[i] + vals[i + 1]
                               for i in range(0, len(vals) - 1, 2)]
                        if len(vals) % 2:
                            nxt.append(vals[-1])
                        vals = nxt
                    plsc.addupdate(acc.at[smin, sl], vals[0])
                plsc.addupdate(cnt.at[smin, pl.ds(0, LANES)], sixteen16)

            @pl.when(smin != smax)
            def _boundary():
                for k in range(LANES):
                    s = svec[k]
                    for j in range(NVREG):
                        sl = pl.ds(j * LANES, LANES)
                        plsc.addupdate(acc.at[s, sl], xbuf[r0 + k, sl])
                    plsc.addupdate(cnt.at[s, pl.ds(0, LANES)], ones16)

        @pl.when(p + 2 < nblocks)
        def _prefetch():
            x_copy(p + 2, base).start()
            b_copy(p + 2, base).start()

    pltpu.make_async_copy(acc, psum_hbm.at[wid], semo).start()
    pltpu.make_async_copy(cnt, pcnt_hbm.at[wid], semo).start()
    pltpu.make_async_copy(acc, psum_hbm.at[wid], semo).wait()
    pltpu.make_async_copy(cnt, pcnt_hbm.at[wid], semo).wait()


def _tc_body(batch_ref, x_ref, sum_ref, cnt_ref):
    i = pl.program_id(0)

    @pl.when(i == 0)
    def _init():
        sum_ref[...] = jnp.zeros_like(sum_ref)
        cnt_ref[...] = jnp.zeros_like(cnt_ref)

    b = batch_ref[0, 0, :]
    onehot = (b[:, None] == jax.lax.broadcasted_iota(
        jnp.int32, (R_TC, NUM_SEG), 1)).astype(jnp.bfloat16)
    xb = x_ref[...].astype(jnp.bfloat16)
    sum_ref[...] += jax.lax.dot_general(
        onehot, xb, (((0,), (0,)), ((), ())),
        preferred_element_type=jnp.float32)
    cnt_ref[...] += jax.lax.dot_general(
        onehot, jnp.ones((R_TC, 128), jnp.bfloat16),
        (((0,), (0,)), ((), ())),
        preferred_element_type=jnp.float32)


def _tc_partial(x, batch3):
    return pl.pallas_call(
        _tc_body,
        grid=(GRID_TC,),
        in_specs=[
            pl.BlockSpec((1, 1, R_TC), lambda i: (i + SKIP_TC, 0, 0)),
            pl.BlockSpec((R_TC, D), lambda i: (i + SKIP_TC, 0)),
        ],
        out_specs=[
            pl.BlockSpec((NUM_SEG, D), lambda i: (0, 0)),
            pl.BlockSpec((NUM_SEG, 128), lambda i: (0, 0)),
        ],
        out_shape=[
            jax.ShapeDtypeStruct((NUM_SEG, D), jnp.float32),
            jax.ShapeDtypeStruct((NUM_SEG, 128), jnp.float32),
        ],
        compiler_params=pltpu.CompilerParams(
            dimension_semantics=("arbitrary",)),
    )(batch3, x)


def _combine_body(ps_ref, pc_ref, ts_ref, tcnt_ref, out_ref):
    sums = jnp.sum(ps_ref[...], axis=0) + ts_ref[...]
    counts = jnp.sum(pc_ref[...], axis=0)[:, 0:1] + tcnt_ref[:, 0:1]
    out_ref[...] = sums / jnp.clip(counts, 1.0, None)


def _tc_combine(psum, pcnt, tsum, tcnt):
    return pl.pallas_call(
        _combine_body,
        out_shape=jax.ShapeDtypeStruct((NUM_SEG, D), jnp.float32),
    )(psum, pcnt, tsum, tcnt)


@jax.jit
def kernel(x, batch):
    batch32 = batch.astype(jnp.int32)
    batch3 = batch32.reshape(N // R_TC, 1, R_TC)
    tsum, tcnt = _tc_partial(x, batch3)
    psum, pcnt = _sc_seg_sum(x, batch32)
    return _tc_combine(psum, pcnt, tsum, tcnt)
